# baseline (speedup 1.0000x reference)
import jax
import jax.numpy as jnp
from jax import lax
from jax.experimental import pallas as pl
from jax.experimental.pallas import tpu as pltpu
from jax.experimental.pallas import tpu_sc as plsc

N = 10000
E = 320000
D_IN = 128
D_H = 128
D_OUT = 64

NC = 2
NS = 16
P = 8
PASSES = P // NC
RR = 1280
N_PAD = P * RR
Z = N
RPT = RR // NS
EPT = E // NS
LS = 2000
NSCAN = EPT // LS
G16 = LS // 16
CH = 128
CSZ = 20224
MAXCH = CSZ // CH


def _sc_agg(with_deg):
    mesh = plsc.VectorSubcoreMesh(core_axis_name="c", subcore_axis_name="s",
                                  num_cores=NC)
    out_type = [jax.ShapeDtypeStruct((N_PAD, D_H), jnp.float32)]
    scratch = [
        pltpu.VMEM((LS,), jnp.int32),
        pltpu.VMEM((LS,), jnp.int32),
        pltpu.VMEM((CSZ,), jnp.int32),
        pltpu.VMEM((CSZ,), jnp.int32),
        pltpu.VMEM((MAXCH, CH), jnp.int32),
        pltpu.VMEM((2, CH, D_H), jnp.float32),
        pltpu.VMEM_SHARED((RR, D_H), jnp.float32),
        pltpu.VMEM_SHARED((NS, CH), jnp.int32),
        pltpu.SemaphoreType.DMA,
    ]
    if with_deg:
        out_type.append(jax.ShapeDtypeStruct((P, NS, RR), jnp.float32))
        scratch.append(pltpu.VMEM((RR + 16,), jnp.float32))

    def body(*refs):
        if with_deg:
            (feat, src_h, dst_h, zrows,
             out_acc, out_deg,
             srcb, dstb, csrc, cdst, dj2d, rows, acc_sh, sidx, sem, degv) = refs
        else:
            (feat, src_h, dst_h, zrows,
             out_acc,
             srcb, dstb, csrc, cdst, dj2d, rows, acc_sh, sidx, sem) = refs
        c = lax.axis_index("c")
        s = lax.axis_index("s")

        @pl.loop(0, PASSES)
        def _range_pass(p):
            r = c * PASSES + p
            base_r = r * RR

            pltpu.sync_copy(zrows, rows.at[0])
            pltpu.sync_copy(rows.at[0, pl.ds(0, RPT)],
                            acc_sh.at[pl.ds(s * RPT, RPT)])
            if with_deg:
                for k in range(RR // 16 + 1):
                    degv[pl.ds(k * 16, 16)] = jnp.zeros((16,), jnp.float32)
            plsc.subcore_barrier()

            @pl.loop(0, NSCAN, init_carry=jnp.int32(0))
            def _scan(q, fill):
                eoff = s * EPT + q * LS
                pltpu.sync_copy(src_h.at[pl.ds(eoff, LS)], srcb)
                pltpu.sync_copy(dst_h.at[pl.ds(eoff, LS)], dstb)
                for k in range(G16):
                    s16 = srcb[pl.ds(k * 16, 16)]
                    d16 = dstb[pl.ds(k * 16, 16)]
                    d2 = d16 - base_r
                    inr = (d2 >= 0) & (d2 < RR)
                    plsc.store_compressed(csrc.at[pl.ds(fill, 16)], s16, mask=inr)
                    plsc.store_compressed(cdst.at[pl.ds(fill, 16)], d2, mask=inr)
                    if with_deg:
                        lane = lax.iota(jnp.int32, 16)
                        d2c = jnp.where(inr, d2, RR + lane)
                        plsc.addupdate_scatter(
                            degv, [d2c], inr.astype(jnp.float32))
                    fill = fill + jnp.sum(inr.astype(jnp.int32))
                return fill

            fill = _scan
            for t in range(CH // 16):
                csrc[pl.ds(fill + t * 16, 16)] = jnp.full((16,), Z, jnp.int32)
                cdst[pl.ds(fill + t * 16, 16)] = jnp.zeros((16,), jnp.int32)
            nch = lax.div(fill + (CH - 1), CH)

            @pl.loop(0, nch)
            def _stage(j):
                pltpu.sync_copy(cdst.at[pl.ds(j * CH, CH)], sidx.at[s])
                pltpu.sync_copy(sidx.at[s], dj2d.at[j])

            @pl.when(nch > 0)
            def _prologue():
                pltpu.async_copy(feat.at[csrc.at[pl.ds(0, CH)]],
                                 rows.at[0], sem)

            @pl.loop(0, nch)
            def _gs(j):
                @pl.when(j + 1 < nch)
                def _fire():
                    pltpu.async_copy(
                        feat.at[csrc.at[pl.ds((j + 1) * CH, CH)]],
                        rows.at[(j + 1) % 2], sem)
                pltpu.make_async_copy(feat.at[csrc.at[pl.ds(0, CH)]],
                                      rows.at[j % 2], sem).wait()
                pltpu.sync_copy(rows.at[j % 2], acc_sh.at[dj2d.at[j]],
                                add=True)

            plsc.subcore_barrier()

            pltpu.sync_copy(acc_sh.at[pl.ds(s * RPT, RPT)],
                            rows.at[0, pl.ds(0, RPT)])
            pltpu.sync_copy(rows.at[0, pl.ds(0, RPT)],
                            out_acc.at[pl.ds(base_r + s * RPT, RPT)])
            if with_deg:
                pltpu.sync_copy(degv.at[pl.ds(0, RR)], out_deg.at[r, s])
            plsc.subcore_barrier()

    return pl.kernel(
        body, out_type=out_type, mesh=mesh, scratch_types=scratch,
        compiler_params=pltpu.CompilerParams(needs_layout_passes=False))


_B = RR


def _tc1_body(acc_ref, deg_ref, x_ref, Wl0_ref, Wr0_ref, Ws0_ref, b0_ref,
              Wr1_ref, b1_ref, h_ref, hr_ref, dinv_ref):
    deg = jnp.sum(deg_ref[0], axis=0)[:, None]
    dinv = 1.0 / jnp.maximum(deg, 1.0)
    mean = acc_ref[...] * dinv
    xb = x_ref[...]
    h = (jnp.dot(mean, Wl0_ref[...], preferred_element_type=jnp.float32)
         + jnp.dot(xb, Wr0_ref[...] + Ws0_ref[...],
                   preferred_element_type=jnp.float32)
         + b0_ref[...])
    h = jnp.maximum(h, 0.0)
    grow = pl.program_id(0) * _B + lax.broadcasted_iota(jnp.int32, (_B, 1), 0)
    h = jnp.where(grow < N, h, 0.0)
    h_ref[...] = h
    hr_ref[...] = (jnp.dot(h, Wr1_ref[...], preferred_element_type=jnp.float32)
                   + b1_ref[...])
    dinv_ref[...] = jnp.broadcast_to(dinv, (_B, D_OUT))


def _tc2_body(acc_ref, Wl1_ref, dinv_ref, hr_ref, out_ref):
    out_ref[...] = (jnp.dot(acc_ref[...], Wl1_ref[...],
                            preferred_element_type=jnp.float32)
                    * dinv_ref[...] + hr_ref[...])


def _tc1(acc0, deg0, xpad, Wl0, Wr0, Ws0, b0, Wr1, b1):
    full = lambda shape: pl.BlockSpec(shape, lambda i: (0,) * len(shape))
    return pl.pallas_call(
        _tc1_body,
        grid=(N_PAD // _B,),
        in_specs=[
            pl.BlockSpec((_B, D_IN), lambda i: (i, 0)),
            pl.BlockSpec((1, NS, RR), lambda i: (i, 0, 0)),
            pl.BlockSpec((_B, D_IN), lambda i: (i, 0)),
            full((D_IN, D_H)), full((D_IN, D_H)), full((D_IN, D_H)),
            full((1, D_H)),
            full((D_H, D_OUT)), full((1, D_OUT)),
        ],
        out_specs=[
            pl.BlockSpec((_B, D_H), lambda i: (i, 0)),
            pl.BlockSpec((_B, D_OUT), lambda i: (i, 0)),
            pl.BlockSpec((_B, D_OUT), lambda i: (i, 0)),
        ],
        out_shape=[
            jax.ShapeDtypeStruct((N_PAD, D_H), jnp.float32),
            jax.ShapeDtypeStruct((N_PAD, D_OUT), jnp.float32),
            jax.ShapeDtypeStruct((N_PAD, D_OUT), jnp.float32),
        ],
    )(acc0, deg0, xpad, Wl0, Wr0, Ws0, b0, Wr1, b1)


def _tc2(acc1, Wl1, dinv, hr):
    full = lambda shape: pl.BlockSpec(shape, lambda i: (0,) * len(shape))
    return pl.pallas_call(
        _tc2_body,
        grid=(N_PAD // _B,),
        in_specs=[
            pl.BlockSpec((_B, D_H), lambda i: (i, 0)),
            full((D_H, D_OUT)),
            pl.BlockSpec((_B, D_OUT), lambda i: (i, 0)),
            pl.BlockSpec((_B, D_OUT), lambda i: (i, 0)),
        ],
        out_specs=pl.BlockSpec((_B, D_OUT), lambda i: (i, 0)),
        out_shape=jax.ShapeDtypeStruct((N_PAD, D_OUT), jnp.float32),
    )(acc1, Wl1, dinv, hr)


@jax.jit
def kernel(x, edge_index, Wl0, Wr0, b0, Ws0, Wl1, Wr1, b1):
    src = edge_index[0]
    dst = edge_index[1]
    xpad = jnp.concatenate(
        [x, jnp.zeros((N_PAD - N, D_IN), jnp.float32)], axis=0)
    zrows = jnp.zeros((CH, D_H), jnp.float32)

    acc0, deg0 = _sc_agg(True)(xpad, src, dst, zrows)
    h, hr, dinv = _tc1(acc0, deg0, xpad, Wl0, Wr0, Ws0,
                       b0.reshape(1, D_H), Wr1, b1.reshape(1, D_OUT))
    res = _sc_agg(False)(h, src, dst, zrows)
    acc1 = res[0] if isinstance(res, (list, tuple)) else res
    out = _tc2(acc1, Wl1, dinv, hr)
    return out[:N]
